# baseline (speedup 1.0000x reference)
import functools
import math

import jax
import jax.numpy as jnp
from jax import lax
from jax.experimental import pallas as pl
from jax.experimental.pallas import tpu as pltpu
from jax.experimental.pallas import tpu_sc as plsc

D_MODEL = 128
MAX_SEQ_LEN = 2048
SCALE = math.sqrt(float(D_MODEL))

_NUM_CORES = 2
_NUM_SUBCORES = 16
_NW = _NUM_CORES * _NUM_SUBCORES
_B = 4 * 2048
_BPW = _B // _NW
_IDX_CHUNK = 128
_NCHUNK = _BPW // _IDX_CHUNK


def _pos_encoding(max_seq_len, d_model):
    position = jnp.arange(0, max_seq_len, dtype=jnp.float32)[:, None]
    div_term = jnp.exp(
        jnp.arange(0, d_model, 2, dtype=jnp.float32)
        * -(math.log(10000.0) / d_model)
    )
    enc = jnp.zeros((max_seq_len, d_model), dtype=jnp.float32)
    enc = enc.at[:, 0::2].set(jnp.sin(position * div_term))
    enc = enc.at[:, 1::2].set(jnp.cos(position * div_term))
    return enc


_MESH = plsc.VectorSubcoreMesh(core_axis_name="c", subcore_axis_name="s")


@functools.partial(
    pl.kernel,
    out_type=jax.ShapeDtypeStruct((_B, D_MODEL), jnp.float32),
    mesh=_MESH,
    scratch_types=[
        pltpu.VMEM((_NCHUNK, _IDX_CHUNK), jnp.int32),
        pltpu.VMEM((_BPW, D_MODEL), jnp.float32),
        pltpu.VMEM((_BPW, D_MODEL), jnp.float32),
        pltpu.SemaphoreType.DMA,
        pltpu.SemaphoreType.DMA,
    ],
)
def _emb_kernel(x_hbm, table_hbm, pe_hbm, out_hbm, idx_v, rows_v, pe_v, sem_g, sem_p):
    wid = lax.axis_index("s") * _NUM_CORES + lax.axis_index("c")
    base = wid * _BPW
    pltpu.sync_copy(x_hbm.at[wid], idx_v)
    cp0 = pltpu.async_copy(
        table_hbm.at[idx_v.at[0]], rows_v.at[pl.ds(0, _IDX_CHUNK)], sem_g
    )
    cp1 = pltpu.async_copy(
        table_hbm.at[idx_v.at[1]], rows_v.at[pl.ds(_IDX_CHUNK, _IDX_CHUNK)], sem_g
    )
    pos_base = lax.rem(base, MAX_SEQ_LEN)
    cpp = pltpu.async_copy(pe_hbm.at[pl.ds(pos_base, _BPW)], pe_v, sem_p)
    cp0.wait()
    cp1.wait()
    cpp.wait()

    def body(i, carry):
        for j in range(D_MODEL // 16):
            sl = pl.ds(j * 16, 16)
            rows_v[i, sl] = rows_v[i, sl] * SCALE + pe_v[i, sl]
        return carry

    lax.fori_loop(0, _BPW, body, 0, unroll=False)

    pltpu.sync_copy(rows_v, out_hbm.at[pl.ds(base, _BPW)])


def kernel(x, table):
    b, s = x.shape
    pe = _pos_encoding(MAX_SEQ_LEN, D_MODEL)
    x_flat = x.reshape(_NW, _NCHUNK, _IDX_CHUNK).astype(jnp.int32)
    out = _emb_kernel(x_flat, table, pe)
    return out.reshape(b, s, D_MODEL)
